# Initial kernel scaffold; baseline (speedup 1.0000x reference)
import jax
import jax.numpy as jnp
from jax.experimental import pallas as pl


def kernel(x, lut):
    raise NotImplementedError("write your pallas kernel here")


# speedup vs baseline: 1.8460x; 1.8460x over previous
import functools

import jax
import jax.numpy as jnp
from jax import lax
from jax.experimental import pallas as pl
from jax.experimental.pallas import tpu as pltpu
from jax.experimental.pallas import tpu_sc as plsc

CHAR = 1000000
DIM = 64
ROWS = 16384
COLS = 50
TOTAL = ROWS * COLS

NC = 2
NS = 16
NW = NC * NS

N_PER_W = TOTAL // NW
IDXW = 128
CHUNK = 1024
K = CHUNK // IDXW
N_CHUNKS = N_PER_W // CHUNK


def _emb_body(x_hbm, lut_hbm, out_hbm, idx_v, rows_v, sem):
    wid = lax.axis_index("s") * NC + lax.axis_index("c")
    row0 = wid * (N_PER_W // IDXW)

    def body(i, carry):
        base = wid * N_PER_W + i * CHUNK
        pltpu.sync_copy(x_hbm.at[pl.ds(row0 + i * K, K)], idx_v)
        copies = [
            pltpu.async_copy(
                lut_hbm.at[idx_v.at[j]],
                rows_v.at[pl.ds(j * IDXW, IDXW)],
                sem,
            )
            for j in range(K)
        ]
        for c in copies:
            c.wait()
        pltpu.sync_copy(rows_v, out_hbm.at[pl.ds(base, CHUNK)])
        return carry

    lax.fori_loop(0, N_CHUNKS, body, 0)


@functools.partial(jax.jit, static_argnames=())
def kernel(x, lut):
    x_flat = x.reshape(TOTAL // IDXW, IDXW).astype(jnp.int32)
    mesh = plsc.VectorSubcoreMesh(core_axis_name="c", subcore_axis_name="s")
    out = pl.kernel(
        _emb_body,
        out_type=jax.ShapeDtypeStruct((TOTAL, DIM), jnp.float32),
        mesh=mesh,
        compiler_params=pltpu.CompilerParams(use_tc_tiling_on_sc=False),
        scratch_types=[
            pltpu.VMEM((K, IDXW), jnp.int32),
            pltpu.VMEM((CHUNK, DIM), jnp.float32),
            pltpu.SemaphoreType.DMA,
        ],
    )(x_flat, lut)
    return out.reshape(ROWS, COLS, DIM)


# speedup vs baseline: 1.8733x; 1.0148x over previous
import functools

import jax
import jax.numpy as jnp
from jax import lax
from jax.experimental import pallas as pl
from jax.experimental.pallas import tpu as pltpu
from jax.experimental.pallas import tpu_sc as plsc

CHAR = 1000000
DIM = 64
ROWS = 16384
COLS = 50
TOTAL = ROWS * COLS

NC = 2
NS = 16
NW = NC * NS

N_PER_W = TOTAL // NW
IDXW = 128
IDX_ROWS = N_PER_W // IDXW
CHUNK = 640
K = CHUNK // IDXW
N_CHUNKS = N_PER_W // CHUNK
NB = 2


def _emb_body(x_hbm, lut_hbm, out_hbm, idx_v, rb0, rb1, sg0, sg1, so0, so1):
    rbufs = (rb0, rb1)
    gsems = (sg0, sg1)
    osems = (so0, so1)
    wid = lax.axis_index("s") * NC + lax.axis_index("c")
    base = wid * N_PER_W

    pltpu.sync_copy(x_hbm.at[pl.ds(wid * IDX_ROWS, IDX_ROWS)], idx_v)

    def fire_gathers(i, b):
        for j in range(K):
            pltpu.async_copy(
                lut_hbm.at[idx_v.at[i * K + j]],
                rbufs[b].at[pl.ds(j * IDXW, IDXW)],
                gsems[b],
            )

    def wait_gathers(b):
        pltpu.make_async_copy(out_hbm.at[pl.ds(0, CHUNK)], rbufs[b], gsems[b]).wait()

    def fire_out(i, b):
        pltpu.async_copy(rbufs[b], out_hbm.at[pl.ds(base + i * CHUNK, CHUNK)], osems[b])

    def wait_out(b):
        pltpu.make_async_copy(rbufs[b], out_hbm.at[pl.ds(0, CHUNK)], osems[b]).wait()

    for b in range(NB):
        fire_gathers(b, b)

    def loop_body(g, carry):
        for b in range(NB):
            i = g * NB + b
            wait_gathers(b)
            fire_out(i, b)
            nxt = i + NB

            @pl.when(nxt < N_CHUNKS)
            def _():
                wait_out(b)
                fire_gathers(nxt, b)

        return carry

    lax.fori_loop(0, N_CHUNKS // NB, loop_body, 0)
    for b in range(NB):
        wait_out(b)


@functools.partial(jax.jit, static_argnames=())
def kernel(x, lut):
    x_flat = x.reshape(TOTAL // IDXW, IDXW).astype(jnp.int32)
    mesh = plsc.VectorSubcoreMesh(core_axis_name="c", subcore_axis_name="s")
    out = pl.kernel(
        _emb_body,
        out_type=jax.ShapeDtypeStruct((TOTAL, DIM), jnp.float32),
        mesh=mesh,
        compiler_params=pltpu.CompilerParams(use_tc_tiling_on_sc=False),
        scratch_types=[
            pltpu.VMEM((IDX_ROWS, IDXW), jnp.int32),
            pltpu.VMEM((CHUNK, DIM), jnp.float32),
            pltpu.VMEM((CHUNK, DIM), jnp.float32),
            pltpu.SemaphoreType.DMA,
            pltpu.SemaphoreType.DMA,
            pltpu.SemaphoreType.DMA,
            pltpu.SemaphoreType.DMA,
        ],
    )(x_flat, lut)
    return out.reshape(ROWS, COLS, DIM)
